# baseline (speedup 1.0000x reference)
import jax
import jax.numpy as jnp
from jax.experimental import pallas as pl
from jax.experimental.pallas import tpu as pltpu

_BATCH = 4
_SEQ = 2048
_D = 1024
_POOL = 64
_LEN = 5
_K = 8
_TAU = 0.5
_HEAD = _K * _LEN
_DC = 512
_NJ = _D // _DC


def _body(x_ref, q_ref, pk_ref, p_ref,
          loss_ref, bkn_ref, out_ref, idx_ref, sp_ref):
    b = pl.program_id(0)
    j = pl.program_id(1)

    @pl.when(jnp.logical_and(b == 0, j == 0))
    def _routing():
        q = q_ref[...]
        qn = q / (jnp.sqrt(jnp.sum(q * q, axis=1, keepdims=True)) + 1e-12)
        pk = pk_ref[...]
        kn = pk / (jnp.sqrt(jnp.sum(pk * pk, axis=1, keepdims=True)) + 1e-12)

        sim = jax.lax.dot_general(
            qn, kn, (((1,), (1,)), ((), ())),
            preferred_element_type=jnp.float32)

        z = sim * (1.0 / _TAU)
        z = z - jnp.max(z, axis=1, keepdims=True)
        ez = jnp.exp(z)
        sp_ref[...] = ez / jnp.sum(ez, axis=1, keepdims=True)

        col = jax.lax.broadcasted_iota(jnp.int32, (_BATCH, _POOL), 1)
        neg_inf = jnp.float32(-jnp.inf)
        work = sim
        masks = []
        iv_all = None
        for k in range(_K):
            m = jnp.max(work, axis=1, keepdims=True)
            cand = jnp.where(work == m, col, _POOL)
            ivk = jnp.min(cand, axis=1, keepdims=True)
            mk = col == ivk
            work = jnp.where(mk, neg_inf, work)
            masks.append(mk)
            iv_all = ivk if iv_all is None else jnp.concatenate([iv_all, ivk], axis=1)

        acc = jnp.zeros((_BATCH, _D), jnp.float32)
        for k in range(_K):
            mf = masks[k].astype(jnp.float32)
            bk = jax.lax.dot_general(
                mf, kn, (((1,), (0,)), ((), ())),
                preferred_element_type=jnp.float32)
            bkn_ref[:, k, :] = bk
            acc = acc + bk
        total = jnp.sum(acc * qn)
        loss_ref[0, 0] = 1.0 - total / _BATCH

        for bb in range(_BATCH):
            for k in range(_K):
                idx_ref[bb, k] = iv_all[bb, k]

    for k in range(_K):
        iv = idx_ref[b, k]
        out_ref[0, k * _LEN:(k + 1) * _LEN, :] = p_ref[iv]

    out_ref[0, _HEAD:, :] = x_ref[0]


def kernel(x_embed, prompt, prompt_key):
    query = x_embed[:, 0, :]

    out_shapes = (
        jax.ShapeDtypeStruct((1, 1), jnp.float32),
        jax.ShapeDtypeStruct((_BATCH, _K, _D), jnp.float32),
        jax.ShapeDtypeStruct((_BATCH, _HEAD + _SEQ, _D), jnp.float32),
        jax.ShapeDtypeStruct((_BATCH, _K), jnp.int32),
        jax.ShapeDtypeStruct((_BATCH, _POOL), jnp.float32),
    )
    smem = pltpu.MemorySpace.SMEM
    loss, bkn, prompted, idx, soft_probs = pl.pallas_call(
        _body,
        grid=(_BATCH, _NJ),
        out_shape=out_shapes,
        in_specs=[
            pl.BlockSpec((1, _SEQ, _DC), lambda b, j: (b, 0, j)),
            pl.BlockSpec((_BATCH, _D), lambda b, j: (0, 0)),
            pl.BlockSpec((_POOL, _D), lambda b, j: (0, 0)),
            pl.BlockSpec((_POOL, _LEN, _DC), lambda b, j: (0, 0, j)),
        ],
        out_specs=[
            pl.BlockSpec((1, 1), lambda b, j: (0, 0), memory_space=smem),
            pl.BlockSpec((_BATCH, _K, _D), lambda b, j: (0, 0, 0)),
            pl.BlockSpec((1, _HEAD + _SEQ, _DC), lambda b, j: (b, 0, j)),
            pl.BlockSpec((_BATCH, _K), lambda b, j: (0, 0), memory_space=smem),
            pl.BlockSpec((_BATCH, _POOL), lambda b, j: (0, 0)),
        ],
    )(x_embed, query, prompt_key, prompt)

    return (loss[0, 0], bkn, prompted, idx, soft_probs)


# speedup vs baseline: 1.1307x; 1.1307x over previous
import jax
import jax.numpy as jnp
from jax.experimental import pallas as pl
from jax.experimental.pallas import tpu as pltpu

_BATCH = 4
_SEQ = 2048
_D = 1024
_POOL = 64
_LEN = 5
_K = 8
_TAU = 0.5
_HEAD = _K * _LEN
_DC = 512
_NJ = _D // _DC


def _body(x_ref, q_ref, pk_ref, p_ref,
          loss_ref, bkn_ref, out_ref, idx_ref, sp_ref):
    b = pl.program_id(0)

    @pl.when(b == 0)
    def _routing():
        q = q_ref[...]
        qn = q / (jnp.sqrt(jnp.sum(q * q, axis=1, keepdims=True)) + 1e-12)
        pk = pk_ref[...]
        kn = pk / (jnp.sqrt(jnp.sum(pk * pk, axis=1, keepdims=True)) + 1e-12)

        sim = jax.lax.dot_general(
            qn, kn, (((1,), (1,)), ((), ())),
            preferred_element_type=jnp.float32)

        z = sim * (1.0 / _TAU)
        z = z - jnp.max(z, axis=1, keepdims=True)
        ez = jnp.exp(z)
        sp_ref[...] = ez / jnp.sum(ez, axis=1, keepdims=True)

        col = jax.lax.broadcasted_iota(jnp.int32, (_BATCH, _POOL), 1)
        neg_inf = jnp.float32(-jnp.inf)
        work = sim
        masks = []
        iv_all = None
        for k in range(_K):
            m = jnp.max(work, axis=1, keepdims=True)
            cand = jnp.where(work == m, col, _POOL)
            ivk = jnp.min(cand, axis=1, keepdims=True)
            mk = col == ivk
            work = jnp.where(mk, neg_inf, work)
            masks.append(mk)
            iv_all = ivk if iv_all is None else jnp.concatenate([iv_all, ivk], axis=1)

        acc = jnp.zeros((_BATCH, _D), jnp.float32)
        for k in range(_K):
            mf = masks[k].astype(jnp.float32)
            bk = jax.lax.dot_general(
                mf, kn, (((1,), (0,)), ((), ())),
                preferred_element_type=jnp.float32)
            bkn_ref[:, k, :] = bk
            acc = acc + bk
        total = jnp.sum(acc * qn)
        loss_ref[0, 0] = 1.0 - total / _BATCH

        for bb in range(_BATCH):
            for k in range(_K):
                idx_ref[bb, k] = iv_all[bb, k]

    for k in range(_K):
        iv = idx_ref[b, k]
        out_ref[0, k * _LEN:(k + 1) * _LEN, :] = p_ref[iv]

    out_ref[0, _HEAD:, :] = x_ref[0]


def kernel(x_embed, prompt, prompt_key):
    query = x_embed[:, 0, :]

    out_shapes = (
        jax.ShapeDtypeStruct((1, 1), jnp.float32),
        jax.ShapeDtypeStruct((_BATCH, _K, _D), jnp.float32),
        jax.ShapeDtypeStruct((_BATCH, _HEAD + _SEQ, _D), jnp.float32),
        jax.ShapeDtypeStruct((_BATCH, _K), jnp.int32),
        jax.ShapeDtypeStruct((_BATCH, _POOL), jnp.float32),
    )
    smem = pltpu.MemorySpace.SMEM
    loss, bkn, prompted, idx, soft_probs = pl.pallas_call(
        _body,
        grid=(_BATCH,),
        out_shape=out_shapes,
        in_specs=[
            pl.BlockSpec((1, _SEQ, _D), lambda b: (b, 0, 0)),
            pl.BlockSpec((_BATCH, _D), lambda b: (0, 0)),
            pl.BlockSpec((_POOL, _D), lambda b: (0, 0)),
            pl.BlockSpec((_POOL, _LEN, _D), lambda b: (0, 0, 0)),
        ],
        out_specs=[
            pl.BlockSpec((1, 1), lambda b: (0, 0), memory_space=smem),
            pl.BlockSpec((_BATCH, _K, _D), lambda b: (0, 0, 0)),
            pl.BlockSpec((1, _HEAD + _SEQ, _D), lambda b: (b, 0, 0)),
            pl.BlockSpec((_BATCH, _K), lambda b: (0, 0), memory_space=smem),
            pl.BlockSpec((_BATCH, _POOL), lambda b: (0, 0)),
        ],
    )(x_embed, query, prompt_key, prompt)

    return (loss[0, 0], bkn, prompted, idx, soft_probs)
